# baseline (speedup 1.0000x reference)
import functools

import jax
import jax.numpy as jnp
from jax import lax
from jax.experimental import pallas as pl
from jax.experimental.pallas import tpu as pltpu
from jax.experimental.pallas import tpu_sc as plsc

_K = 128
_W = _K // 2
_L = 16
_NJ = _K // _L
_NG = _W // _L
_ONE_HOP = 32
_TWO_HOP = 64
_NC = 2
_NS = 16
_NW = _NC * _NS
_NBUF = 4
_CH = 80

_mesh = plsc.VectorSubcoreMesh(
    core_axis_name="c", subcore_axis_name="s",
    num_cores=_NC, num_subcores=_NS)


def _permute(x, idx):
    return lax.gather(
        x, idx[:, None],
        lax.GatherDimensionNumbers(
            offset_dims=(), collapsed_slice_dims=(0,), start_index_map=(0,)),
        (1,), mode=lax.GatherScatterMode.PROMISE_IN_BOUNDS)


def _widen(w):
    lo = lax.bitcast_convert_type(w << 16, jnp.float32)
    hi = lax.bitcast_convert_type(w, jnp.float32)
    return lo, hi


def _load(ref, r):
    out = [None] * _NJ
    for j in range(_NG):
        lo, hi = _widen(ref[r, pl.ds(j * _L, _L)])
        out[j] = lo
        out[_NG + j] = hi
    return out


def _combine_row(xr, dr, wr, g, out_v):
    inv1 = jnp.float32(1.0 / _ONE_HOP)
    inv2 = jnp.float32(1.0 / _TWO_HOP)

    def acc_doc(r, acc):
        return [a + b for a, b in zip(acc, _load(dr, r))]

    def acc_word(r, acc):
        return [a + b for a, b in zip(acc, _load(wr, r))]

    two = lax.fori_loop(1, _TWO_HOP, acc_doc, _load(dr, 0), unroll=8)
    one = lax.fori_loop(1, _ONE_HOP, acc_word, _load(wr, 0), unroll=8)
    x = _load(xr, g)
    t = [x[j] + two[j] * inv2 + one[j] * inv1 for j in range(_NJ)]

    m16 = t[0]
    for j in range(1, _NJ):
        m16 = jnp.maximum(m16, t[j])
    lanes = lax.iota(jnp.int32, _L)
    for k in (8, 4, 2, 1):
        m16 = jnp.maximum(m16, _permute(m16, lanes ^ k))
    e = [jnp.exp(t[j] - m16) for j in range(_NJ)]
    s16 = e[0]
    for j in range(1, _NJ):
        s16 = s16 + e[j]
    for k in (8, 4, 2, 1):
        s16 = s16 + _permute(s16, lanes ^ k)
    r = 1.0 / s16
    for j in range(_NG):
        out_v[g, pl.ds(j * _L, _L)] = e[j] * r
        out_v[g, pl.ds(_W + j * _L, _L)] = e[_NG + j] * r


def _pack_tables(doc_f32, word_f32):
    nd, nw_ = doc_f32.shape[0], word_f32.shape[0]
    nch_d, nch_w = nd // _CH, nw_ // _CH
    half = jnp.uint32(0x8000)

    @functools.partial(
        pl.kernel,
        out_type=(jax.ShapeDtypeStruct((nd, _W), jnp.int32),
                  jax.ShapeDtypeStruct((nw_, _W), jnp.int32)),
        mesh=_mesh,
        compiler_params=pltpu.CompilerParams(use_tc_tiling_on_sc=False),
        scratch_types=[
            pltpu.VMEM((2, _CH, _K), jnp.float32),
            pltpu.VMEM((2, _CH, _W), jnp.int32),
            [pltpu.SemaphoreType.DMA] * 2,
            [pltpu.SemaphoreType.DMA] * 2,
        ],
    )
    def run(doc_hbm, word_hbm, pdoc_hbm, pword_hbm, fbuf, obuf, isems, osems):
        wid = lax.axis_index("s") * _NC + lax.axis_index("c")

        def do_table(src, dst, nch):
            def cid(i):
                return i * _NW + wid

            def issue_in(i, b):
                @pl.when(cid(i) < nch)
                def _():
                    pltpu.async_copy(src.at[pl.ds(cid(i) * _CH, _CH)],
                                     fbuf.at[b], isems[b])

            def wait_in(i, b):
                @pl.when(cid(i) < nch)
                def _():
                    pltpu.make_async_copy(src.at[pl.ds(cid(i) * _CH, _CH)],
                                          fbuf.at[b], isems[b]).wait()

            def wait_out(i, b):
                @pl.when(cid(i) < nch)
                def _():
                    pltpu.make_async_copy(obuf.at[b],
                                          dst.at[pl.ds(cid(i) * _CH, _CH)],
                                          osems[b]).wait()

            niter = nch // _NW + 2
            niter += niter % 2
            issue_in(0, 0)
            issue_in(1, 1)

            def body(p, carry):
                for b in range(2):
                    i = p * 2 + b
                    wait_in(i, b)

                    @pl.when(i >= 2)
                    def _(i=i, b=b):
                        wait_out(i - 2, b)

                    @pl.when(cid(i) < nch)
                    def _(i=i, b=b):
                        ob = obuf.at[b]
                        fb = fbuf.at[b]
                        for r in range(_CH):
                            for j in range(_NG):
                                lo = lax.bitcast_convert_type(
                                    fb[r, pl.ds(j * _L, _L)], jnp.uint32)
                                hi = lax.bitcast_convert_type(
                                    fb[r, pl.ds(_W + j * _L, _L)], jnp.uint32)
                                pk = (((lo + half) >> 16)
                                      | ((hi + half) & jnp.uint32(0xFFFF0000)))
                                ob[r, pl.ds(j * _L, _L)] = (
                                    lax.bitcast_convert_type(pk, jnp.int32))
                        pltpu.async_copy(ob, dst.at[pl.ds(cid(i) * _CH, _CH)],
                                         osems[b])

                    @pl.when(cid(i + 2) < nch)
                    def _(i=i, b=b):
                        issue_in(i + 2, b)
                return carry

            lax.fori_loop(0, niter // 2, body, 0)
            for b in range(2):
                wait_out(niter - 2 + b, b)

        do_table(doc_hbm, pdoc_hbm, nch_d)
        do_table(word_hbm, pword_hbm, nch_w)

    return run(doc_f32, word_f32)


def kernel(v, one_hop_list, two_hop_list, doc_topic_dist, word_topic_dist):
    B = v.shape[0]
    assert B % (_NW * _NBUF) == 0
    ipw = B // _NW

    doc_tab, word_tab = _pack_tables(doc_topic_dist, word_topic_dist)
    v_idx = v.astype(jnp.int32)
    doc_idx = two_hop_list.astype(jnp.int32).reshape(-1)
    word_idx = one_hop_list.astype(jnp.int32).reshape(-1)

    @functools.partial(
        pl.kernel,
        out_type=jax.ShapeDtypeStruct((B, _K), jnp.float32),
        mesh=_mesh,
        compiler_params=pltpu.CompilerParams(use_tc_tiling_on_sc=False),
        scratch_types=[
            pltpu.VMEM((ipw,), jnp.int32),
            pltpu.VMEM((ipw * _TWO_HOP,), jnp.int32),
            pltpu.VMEM((ipw * _ONE_HOP,), jnp.int32),
            pltpu.VMEM((ipw, _W), jnp.int32),
            pltpu.VMEM((_NBUF, _TWO_HOP, _W), jnp.int32),
            pltpu.VMEM((_NBUF, _ONE_HOP, _W), jnp.int32),
            pltpu.VMEM((ipw, _K), jnp.float32),
            [pltpu.SemaphoreType.DMA] * _NBUF,
            [pltpu.SemaphoreType.DMA] * _NBUF,
            pltpu.SemaphoreType.DMA,
        ],
    )
    def run(doc_hbm, word_hbm, vidx_hbm, didx_hbm, widx_hbm, out_hbm,
            vidx_v, didx_v, widx_v, xrows, drows, wrows, out_v,
            dsems, wsems, xsem):
        wid = lax.axis_index("s") * _NC + lax.axis_index("c")
        base = wid * ipw
        pltpu.sync_copy(vidx_hbm.at[pl.ds(base, ipw)], vidx_v)
        pltpu.sync_copy(didx_hbm.at[pl.ds(base * _TWO_HOP, ipw * _TWO_HOP)],
                        didx_v)
        pltpu.sync_copy(widx_hbm.at[pl.ds(base * _ONE_HOP, ipw * _ONE_HOP)],
                        widx_v)
        nx = 128
        cx = [pltpu.async_copy(doc_hbm.at[vidx_v.at[pl.ds(h * nx, nx)]],
                               xrows.at[pl.ds(h * nx, nx)], xsem)
              for h in range(ipw // nx)]
        for c in cx:
            c.wait()

        def issue(g, slot):
            pltpu.async_copy(
                doc_hbm.at[didx_v.at[pl.ds(g * _TWO_HOP, _TWO_HOP)]],
                drows.at[slot], dsems[slot])
            pltpu.async_copy(
                word_hbm.at[widx_v.at[pl.ds(g * _ONE_HOP, _ONE_HOP)]],
                wrows.at[slot], wsems[slot])

        def wait(g, slot):
            pltpu.make_async_copy(
                doc_hbm.at[didx_v.at[pl.ds(g * _TWO_HOP, _TWO_HOP)]],
                drows.at[slot], dsems[slot]).wait()
            pltpu.make_async_copy(
                word_hbm.at[widx_v.at[pl.ds(g * _ONE_HOP, _ONE_HOP)]],
                wrows.at[slot], wsems[slot]).wait()

        for b in range(_NBUF):
            issue(b, b)

        def group(p, carry):
            for b in range(_NBUF):
                g = p * _NBUF + b
                wait(g, b)
                _combine_row(xrows, drows.at[b], wrows.at[b], g, out_v)

                @pl.when(g + _NBUF < ipw)
                def _prefetch(b=b, g=g):
                    issue(g + _NBUF, b)
            return carry

        lax.fori_loop(0, ipw // _NBUF, group, 0)
        pltpu.sync_copy(out_v, out_hbm.at[pl.ds(base, ipw)])

    return run(doc_tab, word_tab, v_idx, doc_idx, word_idx)


# speedup vs baseline: 1.1197x; 1.1197x over previous
import functools

import jax
import jax.numpy as jnp
from jax import lax
from jax.experimental import pallas as pl
from jax.experimental.pallas import tpu as pltpu
from jax.experimental.pallas import tpu_sc as plsc

_K = 128
_W = _K // 2
_L = 16
_NJ = _K // _L
_NG = _W // _L
_ONE_HOP = 32
_TWO_HOP = 64
_NC = 2
_NS = 16
_NW = _NC * _NS
_NBUF = 4
_CH = 50

_mesh = plsc.VectorSubcoreMesh(
    core_axis_name="c", subcore_axis_name="s",
    num_cores=_NC, num_subcores=_NS)


def _permute(x, idx):
    return lax.gather(
        x, idx[:, None],
        lax.GatherDimensionNumbers(
            offset_dims=(), collapsed_slice_dims=(0,), start_index_map=(0,)),
        (1,), mode=lax.GatherScatterMode.PROMISE_IN_BOUNDS)


def _widen(w):
    lo = lax.bitcast_convert_type(w << 16, jnp.float32)
    hi = lax.bitcast_convert_type(w, jnp.float32)
    return lo, hi


def _load(ref, r):
    out = [None] * _NJ
    for j in range(_NG):
        lo, hi = _widen(ref[r, pl.ds(j * _L, _L)])
        out[j] = lo
        out[_NG + j] = hi
    return out


def _combine_row(xr, dr, wr, g, out_v):
    inv1 = jnp.float32(1.0 / _ONE_HOP)
    inv2 = jnp.float32(1.0 / _TWO_HOP)

    def acc_doc(r, acc):
        return [a + b for a, b in zip(acc, _load(dr, r))]

    def acc_word(r, acc):
        return [a + b for a, b in zip(acc, _load(wr, r))]

    two = lax.fori_loop(1, _TWO_HOP, acc_doc, _load(dr, 0), unroll=8)
    one = lax.fori_loop(1, _ONE_HOP, acc_word, _load(wr, 0), unroll=8)
    x = _load(xr, g)
    t = [x[j] + two[j] * inv2 + one[j] * inv1 for j in range(_NJ)]

    m16 = t[0]
    for j in range(1, _NJ):
        m16 = jnp.maximum(m16, t[j])
    lanes = lax.iota(jnp.int32, _L)
    for k in (8, 4, 2, 1):
        m16 = jnp.maximum(m16, _permute(m16, lanes ^ k))
    e = [jnp.exp(t[j] - m16) for j in range(_NJ)]
    s16 = e[0]
    for j in range(1, _NJ):
        s16 = s16 + e[j]
    for k in (8, 4, 2, 1):
        s16 = s16 + _permute(s16, lanes ^ k)
    r = 1.0 / s16
    for j in range(_NG):
        out_v[g, pl.ds(j * _L, _L)] = e[j] * r
        out_v[g, pl.ds(_W + j * _L, _L)] = e[_NG + j] * r


def _pack_tables(doc_f32, word_f32):
    nd, nw_ = doc_f32.shape[0], word_f32.shape[0]
    nch_d, nch_w = nd // _CH, nw_ // _CH
    half = jnp.uint32(0x8000)

    @functools.partial(
        pl.kernel,
        out_type=(jax.ShapeDtypeStruct((nd, _W), jnp.int32),
                  jax.ShapeDtypeStruct((nw_, _W), jnp.int32)),
        mesh=_mesh,
        compiler_params=pltpu.CompilerParams(use_tc_tiling_on_sc=False),
        scratch_types=[
            pltpu.VMEM((2, _CH, _K), jnp.float32),
            pltpu.VMEM((2, _CH, _W), jnp.int32),
            [pltpu.SemaphoreType.DMA] * 2,
            [pltpu.SemaphoreType.DMA] * 2,
        ],
    )
    def run(doc_hbm, word_hbm, pdoc_hbm, pword_hbm, fbuf, obuf, isems, osems):
        wid = lax.axis_index("s") * _NC + lax.axis_index("c")

        def do_table(src, dst, nch):
            def cid(i):
                return i * _NW + wid

            def issue_in(i, b):
                @pl.when(cid(i) < nch)
                def _():
                    pltpu.async_copy(src.at[pl.ds(cid(i) * _CH, _CH)],
                                     fbuf.at[b], isems[b])

            def wait_in(i, b):
                @pl.when(cid(i) < nch)
                def _():
                    pltpu.make_async_copy(src.at[pl.ds(cid(i) * _CH, _CH)],
                                          fbuf.at[b], isems[b]).wait()

            def wait_out(i, b):
                @pl.when(cid(i) < nch)
                def _():
                    pltpu.make_async_copy(obuf.at[b],
                                          dst.at[pl.ds(cid(i) * _CH, _CH)],
                                          osems[b]).wait()

            niter = nch // _NW + 2
            niter += niter % 2
            issue_in(0, 0)
            issue_in(1, 1)

            def body(p, carry):
                for b in range(2):
                    i = p * 2 + b
                    wait_in(i, b)

                    @pl.when(i >= 2)
                    def _(i=i, b=b):
                        wait_out(i - 2, b)

                    @pl.when(cid(i) < nch)
                    def _(i=i, b=b):
                        ob = obuf.at[b]
                        fb = fbuf.at[b]
                        for r in range(_CH):
                            for j in range(_NG):
                                lo = lax.bitcast_convert_type(
                                    fb[r, pl.ds(j * _L, _L)], jnp.uint32)
                                hi = lax.bitcast_convert_type(
                                    fb[r, pl.ds(_W + j * _L, _L)], jnp.uint32)
                                pk = (((lo + half) >> 16)
                                      | ((hi + half) & jnp.uint32(0xFFFF0000)))
                                ob[r, pl.ds(j * _L, _L)] = (
                                    lax.bitcast_convert_type(pk, jnp.int32))
                        pltpu.async_copy(ob, dst.at[pl.ds(cid(i) * _CH, _CH)],
                                         osems[b])

                    @pl.when(cid(i + 2) < nch)
                    def _(i=i, b=b):
                        issue_in(i + 2, b)
                return carry

            lax.fori_loop(0, niter // 2, body, 0)
            for b in range(2):
                wait_out(niter - 2 + b, b)

        do_table(doc_hbm, pdoc_hbm, nch_d)
        do_table(word_hbm, pword_hbm, nch_w)

    return run(doc_f32, word_f32)


def kernel(v, one_hop_list, two_hop_list, doc_topic_dist, word_topic_dist):
    B = v.shape[0]
    assert B % (_NW * _NBUF) == 0
    ipw = B // _NW

    doc_tab, word_tab = _pack_tables(doc_topic_dist, word_topic_dist)
    v_idx = v.astype(jnp.int32)
    doc_idx = two_hop_list.astype(jnp.int32).reshape(-1)
    word_idx = one_hop_list.astype(jnp.int32).reshape(-1)

    @functools.partial(
        pl.kernel,
        out_type=jax.ShapeDtypeStruct((B, _K), jnp.float32),
        mesh=_mesh,
        compiler_params=pltpu.CompilerParams(use_tc_tiling_on_sc=False),
        scratch_types=[
            pltpu.VMEM((ipw,), jnp.int32),
            pltpu.VMEM((ipw * _TWO_HOP,), jnp.int32),
            pltpu.VMEM((ipw * _ONE_HOP,), jnp.int32),
            pltpu.VMEM((ipw, _W), jnp.int32),
            pltpu.VMEM((_NBUF, _TWO_HOP, _W), jnp.int32),
            pltpu.VMEM((_NBUF, _ONE_HOP, _W), jnp.int32),
            pltpu.VMEM((ipw, _K), jnp.float32),
            [pltpu.SemaphoreType.DMA] * _NBUF,
            [pltpu.SemaphoreType.DMA] * _NBUF,
            pltpu.SemaphoreType.DMA,
        ],
    )
    def run(doc_hbm, word_hbm, vidx_hbm, didx_hbm, widx_hbm, out_hbm,
            vidx_v, didx_v, widx_v, xrows, drows, wrows, out_v,
            dsems, wsems, xsem):
        wid = lax.axis_index("s") * _NC + lax.axis_index("c")
        base = wid * ipw
        pltpu.sync_copy(vidx_hbm.at[pl.ds(base, ipw)], vidx_v)
        pltpu.sync_copy(didx_hbm.at[pl.ds(base * _TWO_HOP, ipw * _TWO_HOP)],
                        didx_v)
        pltpu.sync_copy(widx_hbm.at[pl.ds(base * _ONE_HOP, ipw * _ONE_HOP)],
                        widx_v)
        nx = 128
        cx = [pltpu.async_copy(doc_hbm.at[vidx_v.at[pl.ds(h * nx, nx)]],
                               xrows.at[pl.ds(h * nx, nx)], xsem)
              for h in range(ipw // nx)]
        for c in cx:
            c.wait()

        def issue(g, slot):
            pltpu.async_copy(
                doc_hbm.at[didx_v.at[pl.ds(g * _TWO_HOP, _TWO_HOP)]],
                drows.at[slot], dsems[slot])
            pltpu.async_copy(
                word_hbm.at[widx_v.at[pl.ds(g * _ONE_HOP, _ONE_HOP)]],
                wrows.at[slot], wsems[slot])

        def wait(g, slot):
            pltpu.make_async_copy(
                doc_hbm.at[didx_v.at[pl.ds(g * _TWO_HOP, _TWO_HOP)]],
                drows.at[slot], dsems[slot]).wait()
            pltpu.make_async_copy(
                word_hbm.at[widx_v.at[pl.ds(g * _ONE_HOP, _ONE_HOP)]],
                wrows.at[slot], wsems[slot]).wait()

        for b in range(_NBUF):
            issue(b, b)

        def group(p, carry):
            for b in range(_NBUF):
                g = p * _NBUF + b
                wait(g, b)
                _combine_row(xrows, drows.at[b], wrows.at[b], g, out_v)

                @pl.when(g + _NBUF < ipw)
                def _prefetch(b=b, g=g):
                    issue(g + _NBUF, b)
            return carry

        lax.fori_loop(0, ipw // _NBUF, group, 0)
        pltpu.sync_copy(out_v, out_hbm.at[pl.ds(base, ipw)])

    return run(doc_tab, word_tab, v_idx, doc_idx, word_idx)
